# baseline (speedup 1.0000x reference)
import functools

import jax
import jax.numpy as jnp
from jax.experimental import pallas as pl
from jax.experimental.pallas import tpu as pltpu

_BATCH_BLOCK = 8
_CHUNK = 2560

_LN2 = 0.6931471805599453
_NEG_LN_LN2 = 0.36651292058166435


def _threefry_bits(x1):
    ks = (0, 42, (42 ^ 0x1BD11BDA) & 0xFFFFFFFF)
    rot = ((13, 15, 26, 6), (17, 29, 16, 24))
    x0 = x1
    x1 = (x1 << jnp.uint32(13)) | (x1 >> jnp.uint32(19))
    x1 = x0 ^ x1
    for r in rot[0][1:]:
        x0 = x0 + x1
        x1 = (x1 << jnp.uint32(r)) | (x1 >> jnp.uint32(32 - r))
        x1 = x0 ^ x1
    x0 = x0 + jnp.uint32(ks[1])
    x1 = x1 + jnp.uint32((ks[2] + 1) & 0xFFFFFFFF)
    for i in range(1, 5):
        for r in rot[i % 2]:
            x0 = x0 + x1
            x1 = (x1 << jnp.uint32(r)) | (x1 >> jnp.uint32(32 - r))
            x1 = x0 ^ x1
        kx0 = ks[(i + 1) % 3]
        kx1 = (ks[(i + 2) % 3] + i + 1) & 0xFFFFFFFF
        if kx0:
            x0 = x0 + jnp.uint32(kx0)
        x1 = x1 + jnp.uint32(kx1)
    return x0 ^ x1


def _perturbed(x, g, x1, inv):
    a = x * inv + g
    bits = _threefry_bits(x1)
    fbits = (bits >> jnp.uint32(9)) | jnp.uint32(0x3F800000)
    f = jax.lax.bitcast_convert_type(fbits, jnp.float32) - jnp.float32(1.0)
    u = jnp.maximum(f, jnp.float32(1.1754944e-38))
    w2 = -jnp.log2(u)
    return a - jnp.float32(_LN2) * jnp.log2(w2) + jnp.float32(_NEG_LN_LN2)


def _body(t_ref, x_ref, g_ref, o_ref, *, dim, rows, chunk):
    t = t_ref[0]
    inv = jnp.float32(1.0) / (jnp.abs(t) + jnp.float32(1e-8))
    pid = pl.program_id(0)
    nfull = dim // chunk
    tail = dim - nfull * chunk

    rowi = jax.lax.broadcasted_iota(jnp.uint32, (rows, chunk), 0)
    coli = jax.lax.broadcasted_iota(jnp.uint32, (rows, chunk), 1)
    cnt42 = (rowi + (pid * rows).astype(jnp.uint32)) * jnp.uint32(dim) \
        + coli + jnp.uint32(42)
    lane = jax.lax.broadcasted_iota(jnp.int32, (rows, 128), 1)

    def merge(y, off, width, best, bidx):
        for k in range(width // 128):
            yk = y[:, k * 128:(k + 1) * 128]
            upd = yk > best
            best = jnp.where(upd, yk, best)
            bidx = jnp.where(upd, lane + (off + k * 128), bidx)
        return best, bidx

    def loop_body(c, carry):
        best, bidx = carry
        off = c * chunk
        x = x_ref[:, pl.ds(off, chunk)]
        g = g_ref[:, pl.ds(off, chunk)]
        y = _perturbed(x, g, cnt42 + off.astype(jnp.uint32), inv)
        return merge(y, off, chunk, best, bidx)

    best0 = jnp.full((rows, 128), -jnp.inf, jnp.float32)
    bidx0 = jnp.zeros((rows, 128), jnp.int32)
    best, bidx = jax.lax.fori_loop(0, nfull, loop_body, (best0, bidx0))

    if tail:
        tw = ((tail + 127) // 128) * 128
        toff = dim - tw
        xt = x_ref[:, toff:]
        gt = g_ref[:, toff:]
        yt = _perturbed(xt, gt, cnt42[:, :tw] + jnp.uint32(toff), inv)
        best, bidx = merge(yt, toff, tw, best, bidx)

    mrow = jnp.max(best, axis=1, keepdims=True)
    irow = jnp.min(jnp.where(best == mrow, bidx, jnp.int32(dim)), axis=1)

    o_ref[...] = jnp.broadcast_to(irow[:, None], (rows, 128))


@jax.jit
def kernel(logits, guidance_field, temperature):
    if logits.ndim == 1:
        logits = logits[None, :]
    b, d = logits.shape
    rows = _BATCH_BLOCK if b % _BATCH_BLOCK == 0 else b
    grid = b // rows
    gf = guidance_field.reshape(1, d)
    out = pl.pallas_call(
        functools.partial(_body, dim=d, rows=rows, chunk=_CHUNK),
        grid=(grid,),
        in_specs=[
            pl.BlockSpec(memory_space=pltpu.SMEM),
            pl.BlockSpec((rows, d), lambda i: (i, 0)),
            pl.BlockSpec((1, d), lambda i: (0, 0)),
        ],
        out_specs=pl.BlockSpec((rows, 128), lambda i: (i, 0)),
        out_shape=jax.ShapeDtypeStruct((b, 128), jnp.int32),
        compiler_params=pltpu.CompilerParams(
            dimension_semantics=("parallel",)),
    )(temperature, logits, gf)
    return out[:, 0]


# speedup vs baseline: 1.0441x; 1.0441x over previous
import functools

import jax
import jax.numpy as jnp
from jax.experimental import pallas as pl
from jax.experimental.pallas import tpu as pltpu

_BATCH_BLOCK = 8
_CHUNK = 3584

_LN2 = 0.6931471805599453
_NEG_LN_LN2 = 0.36651292058166435


def _threefry_bits(x1):
    ks = (0, 42, (42 ^ 0x1BD11BDA) & 0xFFFFFFFF)
    rot = ((13, 15, 26, 6), (17, 29, 16, 24))
    x0 = x1
    x1 = (x1 << jnp.uint32(13)) | (x1 >> jnp.uint32(19))
    x1 = x0 ^ x1
    for r in rot[0][1:]:
        x0 = x0 + x1
        x1 = (x1 << jnp.uint32(r)) | (x1 >> jnp.uint32(32 - r))
        x1 = x0 ^ x1
    x0 = x0 + jnp.uint32(ks[1])
    x1 = x1 + jnp.uint32((ks[2] + 1) & 0xFFFFFFFF)
    for i in range(1, 5):
        for r in rot[i % 2]:
            x0 = x0 + x1
            x1 = (x1 << jnp.uint32(r)) | (x1 >> jnp.uint32(32 - r))
            x1 = x0 ^ x1
        kx0 = ks[(i + 1) % 3]
        kx1 = (ks[(i + 2) % 3] + i + 1) & 0xFFFFFFFF
        if kx0:
            x0 = x0 + jnp.uint32(kx0)
        x1 = x1 + jnp.uint32(kx1)
    return x0 ^ x1


def _perturbed(x, g, x1, inv):
    a = x * inv + g
    bits = _threefry_bits(x1)
    fbits = (bits >> jnp.uint32(9)) | jnp.uint32(0x3F800000)
    f = jax.lax.bitcast_convert_type(fbits, jnp.float32) - jnp.float32(1.0)
    u = jnp.maximum(f, jnp.float32(1.1754944e-38))
    w2 = -jnp.log2(u)
    return a - jnp.float32(_LN2) * jnp.log2(w2) + jnp.float32(_NEG_LN_LN2)


def _body(t_ref, x_ref, g_ref, o_ref, *, dim, rows, chunk):
    t = t_ref[0]
    inv = jnp.float32(1.0) / (jnp.abs(t) + jnp.float32(1e-8))
    pid = pl.program_id(0)
    nfull = dim // chunk
    tail = dim - nfull * chunk

    rowi = jax.lax.broadcasted_iota(jnp.uint32, (rows, chunk), 0)
    coli = jax.lax.broadcasted_iota(jnp.uint32, (rows, chunk), 1)
    cnt42 = (rowi + (pid * rows).astype(jnp.uint32)) * jnp.uint32(dim) \
        + coli + jnp.uint32(42)
    lane = jax.lax.broadcasted_iota(jnp.int32, (rows, 128), 1)

    def merge(y, off, width, best, bidx):
        for k in range(width // 128):
            yk = y[:, k * 128:(k + 1) * 128]
            upd = yk > best
            best = jnp.where(upd, yk, best)
            bidx = jnp.where(upd, lane + (off + k * 128), bidx)
        return best, bidx

    def loop_body(c, carry):
        best, bidx = carry
        off = c * chunk
        x = x_ref[:, pl.ds(off, chunk)]
        g = g_ref[:, pl.ds(off, chunk)]
        y = _perturbed(x, g, cnt42 + off.astype(jnp.uint32), inv)
        return merge(y, off, chunk, best, bidx)

    best0 = jnp.full((rows, 128), -jnp.inf, jnp.float32)
    bidx0 = jnp.zeros((rows, 128), jnp.int32)
    best, bidx = jax.lax.fori_loop(0, nfull, loop_body, (best0, bidx0))

    if tail:
        tw = ((tail + 127) // 128) * 128
        toff = dim - tw
        xt = x_ref[:, toff:]
        gt = g_ref[:, toff:]
        yt = _perturbed(xt, gt, cnt42[:, :tw] + jnp.uint32(toff), inv)
        best, bidx = merge(yt, toff, tw, best, bidx)

    mrow = jnp.max(best, axis=1, keepdims=True)
    irow = jnp.min(jnp.where(best == mrow, bidx, jnp.int32(dim)), axis=1)

    o_ref[...] = jnp.broadcast_to(irow[:, None], (rows, 128))


@jax.jit
def kernel(logits, guidance_field, temperature):
    if logits.ndim == 1:
        logits = logits[None, :]
    b, d = logits.shape
    rows = _BATCH_BLOCK if b % _BATCH_BLOCK == 0 else b
    grid = b // rows
    gf = guidance_field.reshape(1, d)
    out = pl.pallas_call(
        functools.partial(_body, dim=d, rows=rows, chunk=_CHUNK),
        grid=(grid,),
        in_specs=[
            pl.BlockSpec(memory_space=pltpu.SMEM),
            pl.BlockSpec((rows, d), lambda i: (i, 0)),
            pl.BlockSpec((1, d), lambda i: (0, 0)),
        ],
        out_specs=pl.BlockSpec((rows, 128), lambda i: (i, 0)),
        out_shape=jax.ShapeDtypeStruct((b, 128), jnp.int32),
        compiler_params=pltpu.CompilerParams(
            dimension_semantics=("parallel",)),
    )(temperature, logits, gf)
    return out[:, 0]


# speedup vs baseline: 1.0790x; 1.0335x over previous
import functools

import jax
import jax.numpy as jnp
from jax.experimental import pallas as pl
from jax.experimental.pallas import tpu as pltpu

_BATCH_BLOCK = 16
_CHUNK = 1536

_LN2 = 0.6931471805599453
_NEG_LN_LN2 = 0.36651292058166435


def _threefry_bits(x1):
    ks = (0, 42, (42 ^ 0x1BD11BDA) & 0xFFFFFFFF)
    rot = ((13, 15, 26, 6), (17, 29, 16, 24))
    x0 = x1
    x1 = (x1 << jnp.uint32(13)) | (x1 >> jnp.uint32(19))
    x1 = x0 ^ x1
    for r in rot[0][1:]:
        x0 = x0 + x1
        x1 = (x1 << jnp.uint32(r)) | (x1 >> jnp.uint32(32 - r))
        x1 = x0 ^ x1
    x0 = x0 + jnp.uint32(ks[1])
    x1 = x1 + jnp.uint32((ks[2] + 1) & 0xFFFFFFFF)
    for i in range(1, 5):
        for r in rot[i % 2]:
            x0 = x0 + x1
            x1 = (x1 << jnp.uint32(r)) | (x1 >> jnp.uint32(32 - r))
            x1 = x0 ^ x1
        kx0 = ks[(i + 1) % 3]
        kx1 = (ks[(i + 2) % 3] + i + 1) & 0xFFFFFFFF
        if kx0:
            x0 = x0 + jnp.uint32(kx0)
        x1 = x1 + jnp.uint32(kx1)
    return x0 ^ x1


def _perturbed(x, g, x1, inv):
    a = x * inv + g
    bits = _threefry_bits(x1)
    fbits = (bits >> jnp.uint32(9)) | jnp.uint32(0x3F800000)
    f = jax.lax.bitcast_convert_type(fbits, jnp.float32) - jnp.float32(1.0)
    u = jnp.maximum(f, jnp.float32(1.1754944e-38))
    w2 = -jnp.log2(u)
    return a - jnp.float32(_LN2) * jnp.log2(w2) + jnp.float32(_NEG_LN_LN2)


def _body(t_ref, x_ref, g_ref, o_ref, *, dim, rows, chunk):
    t = t_ref[0]
    inv = jnp.float32(1.0) / (jnp.abs(t) + jnp.float32(1e-8))
    pid = pl.program_id(0)
    nfull = dim // chunk
    tail = dim - nfull * chunk

    rowi = jax.lax.broadcasted_iota(jnp.uint32, (rows, chunk), 0)
    coli = jax.lax.broadcasted_iota(jnp.uint32, (rows, chunk), 1)
    cnt42 = (rowi + (pid * rows).astype(jnp.uint32)) * jnp.uint32(dim) \
        + coli + jnp.uint32(42)
    lane = jax.lax.broadcasted_iota(jnp.int32, (rows, 128), 1)

    def merge(y, off, width, best, bidx):
        for k in range(width // 128):
            yk = y[:, k * 128:(k + 1) * 128]
            upd = yk > best
            best = jnp.where(upd, yk, best)
            bidx = jnp.where(upd, lane + (off + k * 128), bidx)
        return best, bidx

    def loop_body(c, carry):
        best, bidx = carry
        off = c * chunk
        x = x_ref[:, pl.ds(off, chunk)]
        g = g_ref[:, pl.ds(off, chunk)]
        y = _perturbed(x, g, cnt42 + off.astype(jnp.uint32), inv)
        return merge(y, off, chunk, best, bidx)

    best0 = jnp.full((rows, 128), -jnp.inf, jnp.float32)
    bidx0 = jnp.zeros((rows, 128), jnp.int32)
    best, bidx = jax.lax.fori_loop(0, nfull, loop_body, (best0, bidx0))

    if tail:
        tw = ((tail + 127) // 128) * 128
        toff = dim - tw
        xt = x_ref[:, toff:]
        gt = g_ref[:, toff:]
        yt = _perturbed(xt, gt, cnt42[:, :tw] + jnp.uint32(toff), inv)
        best, bidx = merge(yt, toff, tw, best, bidx)

    mrow = jnp.max(best, axis=1, keepdims=True)
    irow = jnp.min(jnp.where(best == mrow, bidx, jnp.int32(dim)), axis=1)

    o_ref[...] = jnp.broadcast_to(irow[:, None], (rows, 128))


@jax.jit
def kernel(logits, guidance_field, temperature):
    if logits.ndim == 1:
        logits = logits[None, :]
    b, d = logits.shape
    rows = _BATCH_BLOCK if b % _BATCH_BLOCK == 0 else b
    grid = b // rows
    gf = guidance_field.reshape(1, d)
    out = pl.pallas_call(
        functools.partial(_body, dim=d, rows=rows, chunk=_CHUNK),
        grid=(grid,),
        in_specs=[
            pl.BlockSpec(memory_space=pltpu.SMEM),
            pl.BlockSpec((rows, d), lambda i: (i, 0)),
            pl.BlockSpec((1, d), lambda i: (0, 0)),
        ],
        out_specs=pl.BlockSpec((rows, 128), lambda i: (i, 0)),
        out_shape=jax.ShapeDtypeStruct((b, 128), jnp.int32),
        compiler_params=pltpu.CompilerParams(
            dimension_semantics=("parallel",)),
    )(temperature, logits, gf)
    return out[:, 0]
